# Initial kernel scaffold; baseline (speedup 1.0000x reference)
import jax
import jax.numpy as jnp
from jax.experimental import pallas as pl


def kernel(x, edge_index, W1, a_src1, a_dst1, b1, g1, be1, W2, a_src2, a_dst2, b2, g2, be2, Wa, ba):
    raise NotImplementedError("write your pallas kernel here")


# speedup vs baseline: 92.2525x; 92.2525x over previous
import functools

import jax
import jax.numpy as jnp
from jax import lax
from jax.experimental import pallas as pl
from jax.experimental.pallas import tpu as pltpu
from jax.experimental.pallas import tpu_sc as plsc

N = 10000
NP = 10112
SENT = N
HEADS = 2
C = 32
HID = 64
NTILES = 32
K = 128
NCH = 81
EPT = NCH * K
TOT = NTILES * EPT
RPT = NP // 16
NEG = -1e30


def _tc_head_body(din, x_ref, w_ref, asrc_ref, adst_ref,
                  h_ref, tabs_ref, mv_ref):
    x = x_ref[...]
    w = w_ref[...]
    h = jnp.dot(x, w, preferred_element_type=jnp.float32)
    h_ref[...] = h
    av = asrc_ref[...]
    bv = adst_ref[...]
    ps = h * av
    pd = h * bv
    as0 = ps[:, :C].sum(axis=1, keepdims=True)
    as1 = ps[:, C:].sum(axis=1, keepdims=True)
    ad0 = pd[:, :C].sum(axis=1, keepdims=True)
    ad1 = pd[:, C:].sum(axis=1, keepdims=True)
    row = lax.broadcasted_iota(jnp.int32, (NP, 1), 0)
    valid = row < N
    as0 = jnp.where(valid, as0, NEG)
    as1 = jnp.where(valid, as1, NEG)
    ad0 = jnp.where(valid, ad0, NEG)
    ad1 = jnp.where(valid, ad1, NEG)
    tabs_ref[...] = jnp.concatenate([as0, as1, ad0, ad1], axis=1)
    m0 = jnp.max(as0) + jnp.max(ad0)
    m1 = jnp.max(as1) + jnp.max(ad1)
    col = lax.broadcasted_iota(jnp.int32, (1, 32), 1)
    mv_ref[...] = jnp.where(col < 16, m0, m1)


def _tc_head(x_pad, w, asrc, adst, din):
    return pl.pallas_call(
        functools.partial(_tc_head_body, din),
        out_shape=[
            jax.ShapeDtypeStruct((NP, HID), jnp.float32),
            jax.ShapeDtypeStruct((NP, 4), jnp.float32),
            jax.ShapeDtypeStruct((1, 32), jnp.float32),
        ],
    )(x_pad, w, asrc, adst)


def _sc_edge_body(tabs_hbm, mv_hbm, src_hbm, dst_hbm, h_hbm,
                  acc_hbm, den_hbm,
                  tabs_v, sidx_v, didx_v, didxc_v, den_v, rows_v, exb_v, mv_v,
                  acc_sh, sem):
    c = lax.axis_index("c")
    s = lax.axis_index("s")
    wid = c * 16 + s

    pltpu.sync_copy(tabs_hbm, tabs_v)
    pltpu.sync_copy(mv_hbm, mv_v)
    pltpu.sync_copy(src_hbm.at[wid], sidx_v)
    pltpu.sync_copy(dst_hbm.at[wid], didx_v)

    zv = jnp.zeros((16,), jnp.float32)

    def zden(i, carry):
        den_v[pl.ds(i * 16, 16)] = zv
        return carry

    lax.fori_loop(0, (2 * NP) // 16, zden, 0)

    def zrow(i, carry):
        rows_v[i, pl.ds(0, 16)] = zv
        rows_v[i, pl.ds(16, 16)] = zv
        rows_v[i, pl.ds(32, 16)] = zv
        rows_v[i, pl.ds(48, 16)] = zv
        return carry

    lax.fori_loop(0, K, zrow, 0)

    base = s * RPT
    for i in range(8):
        pltpu.sync_copy(rows_v.at[pl.ds(0, 79)],
                        acc_sh.at[pl.ds(base + i * 79, 79)])
    plsc.subcore_barrier()

    m0 = mv_v[pl.ds(0, 16)]
    m1 = mv_v[pl.ds(16, 16)]

    def chunk(j, carry):
        cp = pltpu.async_copy(h_hbm.at[sidx_v.at[j]], rows_v, sem)

        def alpha16(t, inner):
            sv = sidx_v[j, pl.ds(t * 16, 16)]
            dv = didx_v[j, pl.ds(t * 16, 16)]
            didxc_v[pl.ds(t * 16, 16)] = dv
            s4 = sv * 4
            d4 = dv * 4
            as0 = plsc.load_gather(tabs_v, [s4])
            as1 = plsc.load_gather(tabs_v, [s4 + 1])
            ad0 = plsc.load_gather(tabs_v, [d4 + 2])
            ad1 = plsc.load_gather(tabs_v, [d4 + 3])
            a0 = as0 + ad0
            a1 = as1 + ad1
            a0 = jnp.where(a0 > 0.0, a0, a0 * 0.2)
            a1 = jnp.where(a1 > 0.0, a1, a1 * 0.2)
            e0 = jnp.exp(a0 - m0)
            e1 = jnp.exp(a1 - m1)
            d2 = dv * 2
            plsc.addupdate_scatter(den_v, [d2], e0)
            plsc.addupdate_scatter(den_v, [d2 + 1], e1)
            exb_v[pl.ds(t * 16, 16)] = e0
            exb_v[pl.ds(K + t * 16, 16)] = e1
            return inner

        lax.fori_loop(0, 8, alpha16, 0)
        cp.wait()

        def mulrow(r, inner):
            s0 = plsc.load_gather(exb_v, [jnp.full((16,), 0, jnp.int32) + r])
            s1 = plsc.load_gather(exb_v, [jnp.full((16,), K, jnp.int32) + r])
            rows_v[r, pl.ds(0, 16)] = rows_v[r, pl.ds(0, 16)] * s0
            rows_v[r, pl.ds(16, 16)] = rows_v[r, pl.ds(16, 16)] * s0
            rows_v[r, pl.ds(32, 16)] = rows_v[r, pl.ds(32, 16)] * s1
            rows_v[r, pl.ds(48, 16)] = rows_v[r, pl.ds(48, 16)] * s1
            return inner

        lax.fori_loop(0, K, mulrow, 0)
        pltpu.sync_copy(rows_v, acc_sh.at[didxc_v], add=True)
        return carry

    lax.fori_loop(0, NCH, chunk, 0)

    pltpu.sync_copy(den_v, den_hbm.at[wid])
    plsc.subcore_barrier()
    pltpu.sync_copy(acc_sh.at[pl.ds(base, RPT)],
                    acc_hbm.at[c, pl.ds(base, RPT)])


def _sc_edge(tabs, mv, srcp, dstp, h_pad):
    mesh = plsc.VectorSubcoreMesh(core_axis_name="c", subcore_axis_name="s")
    fn = functools.partial(
        pl.kernel,
        mesh=mesh,
        compiler_params=pltpu.CompilerParams(
            needs_layout_passes=False, use_tc_tiling_on_sc=False),
        out_type=[
            jax.ShapeDtypeStruct((2, NP, HID), jnp.float32),
            jax.ShapeDtypeStruct((NTILES, 2 * NP), jnp.float32),
        ],
        scratch_types=[
            pltpu.VMEM((4 * NP,), jnp.float32),
            pltpu.VMEM((NCH, K), jnp.int32),
            pltpu.VMEM((NCH, K), jnp.int32),
            pltpu.VMEM((K,), jnp.int32),
            pltpu.VMEM((2 * NP,), jnp.float32),
            pltpu.VMEM((K, HID), jnp.float32),
            pltpu.VMEM((2 * K,), jnp.float32),
            pltpu.VMEM((32,), jnp.float32),
            pltpu.VMEM_SHARED((NP, HID), jnp.float32),
            pltpu.SemaphoreType.DMA,
        ],
    )(_sc_edge_body)
    return fn(tabs, mv, srcp, dstp, h_pad)


def _denred_body(den_ref, out_ref):
    t = pl.program_id(0)

    @pl.when(t == 0)
    def _():
        out_ref[...] = jnp.zeros_like(out_ref)

    out_ref[...] += den_ref[0]


def _denred(den):
    return pl.pallas_call(
        _denred_body,
        grid=(NTILES,),
        in_specs=[pl.BlockSpec((1, 158, 128), lambda t: (t, 0, 0))],
        out_specs=pl.BlockSpec((158, 128), lambda t: (0, 0)),
        out_shape=jax.ShapeDtypeStruct((158, 128), jnp.float32),
    )(den)


def _combine(acc_ref, den_ref, b_ref, g_ref, be_ref):
    acc = acc_ref[0] + acc_ref[1]
    den = den_ref[...]
    d0 = den[:, 0:1]
    d1 = den[:, 1:2]
    col = lax.broadcasted_iota(jnp.int32, (NP, HID), 1)
    divv = jnp.where(col < C, d0, d1)
    out = acc / (divv + 1e-16) + b_ref[...]
    mu = out.mean(axis=1, keepdims=True)
    var = ((out - mu) ** 2).mean(axis=1, keepdims=True)
    ln = (out - mu) / jnp.sqrt(var + 1e-5) * g_ref[...] + be_ref[...]
    return ln


def _tc_mid_body(acc_ref, den_ref, b_ref, g_ref, be_ref, w2_ref,
                 asrc_ref, adst_ref,
                 x1_ref, h_ref, tabs_ref, mv_ref):
    ln = _combine(acc_ref, den_ref, b_ref, g_ref, be_ref)
    x1 = jnp.where(ln > 0.0, ln, jnp.exp(ln) - 1.0)
    x1_ref[...] = x1
    _tc_head_body(HID, x1_ref, w2_ref, asrc_ref, adst_ref,
                  h_ref, tabs_ref, mv_ref)


def _tc_mid(acc1, den1, b1, g1, be1, w2, asrc2, adst2):
    return pl.pallas_call(
        _tc_mid_body,
        out_shape=[
            jax.ShapeDtypeStruct((NP, HID), jnp.float32),
            jax.ShapeDtypeStruct((NP, HID), jnp.float32),
            jax.ShapeDtypeStruct((NP, 4), jnp.float32),
            jax.ShapeDtypeStruct((1, 32), jnp.float32),
        ],
    )(acc1, den1, b1, g1, be1, w2, asrc2, adst2)


def _tc_fin_body(acc_ref, den_ref, b_ref, g_ref, be_ref, x1_ref, wa_ref,
                 ba_ref, out_ref):
    ln = _combine(acc_ref, den_ref, b_ref, g_ref, be_ref)
    x2 = ln + x1_ref[...]
    x2 = jnp.where(x2 > 0.0, x2, jnp.exp(x2) - 1.0)
    logits = jnp.dot(x2, wa_ref[...], preferred_element_type=jnp.float32)
    logits = logits + ba_ref[...]
    row = lax.broadcasted_iota(jnp.int32, (NP, 1), 0)
    logits = jnp.where(row < N, logits, NEG)
    m = jnp.max(logits)
    wgt = jnp.exp(logits - m)
    z = jnp.sum(wgt)
    attw = wgt / z
    out_ref[...] = jnp.sum(x2 * attw, axis=0, keepdims=True)


def _tc_fin(acc2, den2, b2, g2, be2, x1, wa, ba):
    return pl.pallas_call(
        _tc_fin_body,
        out_shape=jax.ShapeDtypeStruct((1, HID), jnp.float32),
    )(acc2, den2, b2, g2, be2, x1, wa, ba)


def kernel(x, edge_index, W1, a_src1, a_dst1, b1, g1, be1, W2, a_src2,
           a_dst2, b2, g2, be2, Wa, ba):
    x_pad = jnp.pad(x, ((0, NP - N), (0, 0)))
    loop = jnp.arange(N, dtype=jnp.int32)
    padv = jnp.full((TOT - 320000 - N,), SENT, dtype=jnp.int32)
    srcp = jnp.concatenate([edge_index[0].astype(jnp.int32), loop, padv])
    dstp = jnp.concatenate([edge_index[1].astype(jnp.int32), loop, padv])
    srcp = srcp.reshape(NTILES, NCH, K)
    dstp = dstp.reshape(NTILES, NCH, K)

    h1, tabs1, mv1 = _tc_head(x_pad, W1, a_src1.reshape(1, HID),
                              a_dst1.reshape(1, HID), 128)
    acc1, den1 = _sc_edge(tabs1.reshape(4 * NP), mv1.reshape(32), srcp, dstp, h1)
    den1 = _denred(den1.reshape(NTILES, 158, 128)).reshape(NP, 2)
    x1, h2, tabs2, mv2 = _tc_mid(acc1, den1,
                                 b1.reshape(1, HID), g1.reshape(1, HID),
                                 be1.reshape(1, HID), W2,
                                 a_src2.reshape(1, HID),
                                 a_dst2.reshape(1, HID))
    acc2, den2 = _sc_edge(tabs2.reshape(4 * NP), mv2.reshape(32), srcp, dstp, h2)
    den2 = _denred(den2.reshape(NTILES, 158, 128)).reshape(NP, 2)
    pooled = _tc_fin(acc2, den2,
                     b2.reshape(1, HID), g2.reshape(1, HID),
                     be2.reshape(1, HID), x1, Wa, ba.reshape(1, 1))
    return pooled.reshape(HID)


# speedup vs baseline: 116.8553x; 1.2667x over previous
import functools

import jax
import jax.numpy as jnp
from jax import lax
from jax.experimental import pallas as pl
from jax.experimental.pallas import tpu as pltpu
from jax.experimental.pallas import tpu_sc as plsc

N = 10000
NP = 10112
SENT = N
HEADS = 2
C = 32
HID = 64
NTILES = 32
K = 64
NCH = 162
EPT = NCH * K
TOT = NTILES * EPT
RPT = NP // 16
NEG = -1e30


def _tc_head_body(din, x_ref, w_ref, asrc_ref, adst_ref,
                  h_ref, tabs_ref, mv_ref):
    x = x_ref[...]
    w = w_ref[...]
    h = jnp.dot(x, w, preferred_element_type=jnp.float32)
    h_ref[...] = h
    av = asrc_ref[...]
    bv = adst_ref[...]
    ps = h * av
    pd = h * bv
    as0 = ps[:, :C].sum(axis=1, keepdims=True)
    as1 = ps[:, C:].sum(axis=1, keepdims=True)
    ad0 = pd[:, :C].sum(axis=1, keepdims=True)
    ad1 = pd[:, C:].sum(axis=1, keepdims=True)
    row = lax.broadcasted_iota(jnp.int32, (NP, 1), 0)
    valid = row < N
    as0 = jnp.where(valid, as0, NEG)
    as1 = jnp.where(valid, as1, NEG)
    ad0 = jnp.where(valid, ad0, NEG)
    ad1 = jnp.where(valid, ad1, NEG)
    tabs_ref[...] = jnp.concatenate([as0, as1, ad0, ad1], axis=1)
    m0 = jnp.max(as0) + jnp.max(ad0)
    m1 = jnp.max(as1) + jnp.max(ad1)
    col = lax.broadcasted_iota(jnp.int32, (1, 32), 1)
    mv_ref[...] = jnp.where(col < 16, m0, m1)


def _tc_head(x_pad, w, asrc, adst, din):
    return pl.pallas_call(
        functools.partial(_tc_head_body, din),
        out_shape=[
            jax.ShapeDtypeStruct((NP, HID), jnp.float32),
            jax.ShapeDtypeStruct((NP, 4), jnp.float32),
            jax.ShapeDtypeStruct((1, 32), jnp.float32),
        ],
    )(x_pad, w, asrc, adst)


def _sc_edge_body(tabs_hbm, mv_hbm, src_hbm, dst_hbm, h_hbm,
                  acc_hbm, den_hbm,
                  tabs_v, sidx_v, didx_v, didxc_v, den_v, rows_v, exb_v, mv_v,
                  acc_sh, sem, sem2):
    c = lax.axis_index("c")
    s = lax.axis_index("s")
    wid = c * 16 + s

    pltpu.sync_copy(tabs_hbm, tabs_v)
    pltpu.sync_copy(mv_hbm, mv_v)
    pltpu.sync_copy(src_hbm.at[wid], sidx_v)
    pltpu.sync_copy(dst_hbm.at[wid], didx_v)

    zv = jnp.zeros((16,), jnp.float32)

    def zden(i, carry):
        den_v[pl.ds(i * 16, 16)] = zv
        return carry

    lax.fori_loop(0, (2 * NP) // 16, zden, 0)

    def zrow(i, carry):
        rows_v[0, i, pl.ds(0, 16)] = zv
        rows_v[0, i, pl.ds(16, 16)] = zv
        rows_v[0, i, pl.ds(32, 16)] = zv
        rows_v[0, i, pl.ds(48, 16)] = zv
        return carry

    lax.fori_loop(0, K, zrow, 0)

    base = s * RPT
    for i in range(9):
        pltpu.sync_copy(rows_v.at[0, pl.ds(0, 64)],
                        acc_sh.at[pl.ds(base + i * 64, 64)])
    pltpu.sync_copy(rows_v.at[0, pl.ds(0, 56)],
                    acc_sh.at[pl.ds(base + 576, 56)])
    plsc.subcore_barrier()

    m0 = mv_v[pl.ds(0, 16)]
    m1 = mv_v[pl.ds(16, 16)]
    sems = (sem, sem2)

    pltpu.async_copy(h_hbm.at[sidx_v.at[0]], rows_v.at[0], sems[0])

    def chunk(j, carry):
        def alpha16(t, inner):
            sv = sidx_v[j, pl.ds(t * 16, 16)]
            dv = didx_v[j, pl.ds(t * 16, 16)]
            didxc_v[pl.ds(t * 16, 16)] = dv
            s4 = sv * 4
            d4 = dv * 4
            as0 = plsc.load_gather(tabs_v, [s4])
            as1 = plsc.load_gather(tabs_v, [s4 + 1])
            ad0 = plsc.load_gather(tabs_v, [d4 + 2])
            ad1 = plsc.load_gather(tabs_v, [d4 + 3])
            a0 = as0 + ad0
            a1 = as1 + ad1
            a0 = jnp.where(a0 > 0.0, a0, a0 * 0.2)
            a1 = jnp.where(a1 > 0.0, a1, a1 * 0.2)
            e0 = jnp.exp(a0 - m0)
            e1 = jnp.exp(a1 - m1)
            d2 = dv * 2
            plsc.addupdate_scatter(den_v, [d2], e0)
            plsc.addupdate_scatter(den_v, [d2 + 1], e1)
            exb_v[pl.ds(t * 16, 16)] = e0
            exb_v[pl.ds(K + t * 16, 16)] = e1
            return inner

        def halfbody(b):
            @pl.when(j + 1 < NCH)
            def _():
                pltpu.async_copy(h_hbm.at[sidx_v.at[j + 1]],
                                 rows_v.at[1 - b], sems[1 - b])

            lax.fori_loop(0, K // 16, alpha16, 0)
            pltpu.make_async_copy(h_hbm.at[sidx_v.at[j]],
                                  rows_v.at[b], sems[b]).wait()

            def mulrow(r, inner):
                s0 = plsc.load_gather(exb_v,
                                      [jnp.full((16,), 0, jnp.int32) + r])
                s1 = plsc.load_gather(exb_v,
                                      [jnp.full((16,), K, jnp.int32) + r])
                rows_v[b, r, pl.ds(0, 16)] = rows_v[b, r, pl.ds(0, 16)] * s0
                rows_v[b, r, pl.ds(16, 16)] = rows_v[b, r, pl.ds(16, 16)] * s0
                rows_v[b, r, pl.ds(32, 16)] = rows_v[b, r, pl.ds(32, 16)] * s1
                rows_v[b, r, pl.ds(48, 16)] = rows_v[b, r, pl.ds(48, 16)] * s1
                return inner

            lax.fori_loop(0, K, mulrow, 0)
            pltpu.sync_copy(rows_v.at[b], acc_sh.at[didxc_v], add=True)

        @pl.when(j % 2 == 0)
        def _():
            halfbody(0)

        @pl.when(j % 2 == 1)
        def _():
            halfbody(1)

        return carry

    lax.fori_loop(0, NCH, chunk, 0)

    pltpu.sync_copy(den_v, den_hbm.at[wid])
    plsc.subcore_barrier()
    pltpu.sync_copy(acc_sh.at[pl.ds(base, RPT)],
                    acc_hbm.at[c, pl.ds(base, RPT)])


def _sc_edge(tabs, mv, srcp, dstp, h_pad):
    mesh = plsc.VectorSubcoreMesh(core_axis_name="c", subcore_axis_name="s")
    fn = functools.partial(
        pl.kernel,
        mesh=mesh,
        compiler_params=pltpu.CompilerParams(
            needs_layout_passes=False, use_tc_tiling_on_sc=False),
        out_type=[
            jax.ShapeDtypeStruct((2, NP, HID), jnp.float32),
            jax.ShapeDtypeStruct((NTILES, 2 * NP), jnp.float32),
        ],
        scratch_types=[
            pltpu.VMEM((4 * NP,), jnp.float32),
            pltpu.VMEM((NCH, K), jnp.int32),
            pltpu.VMEM((NCH, K), jnp.int32),
            pltpu.VMEM((K,), jnp.int32),
            pltpu.VMEM((2 * NP,), jnp.float32),
            pltpu.VMEM((2, K, HID), jnp.float32),
            pltpu.VMEM((2 * K,), jnp.float32),
            pltpu.VMEM((32,), jnp.float32),
            pltpu.VMEM_SHARED((NP, HID), jnp.float32),
            pltpu.SemaphoreType.DMA,
            pltpu.SemaphoreType.DMA,
        ],
    )(_sc_edge_body)
    return fn(tabs, mv, srcp, dstp, h_pad)


def _denred_body(den_ref, out_ref):
    t = pl.program_id(0)

    @pl.when(t == 0)
    def _():
        out_ref[...] = jnp.zeros_like(out_ref)

    out_ref[...] += den_ref[0]


def _denred(den):
    return pl.pallas_call(
        _denred_body,
        grid=(NTILES,),
        in_specs=[pl.BlockSpec((1, 158, 128), lambda t: (t, 0, 0))],
        out_specs=pl.BlockSpec((158, 128), lambda t: (0, 0)),
        out_shape=jax.ShapeDtypeStruct((158, 128), jnp.float32),
    )(den)


def _combine(acc_ref, den_ref, b_ref, g_ref, be_ref):
    acc = acc_ref[0] + acc_ref[1]
    den = den_ref[...]
    d0 = den[:, 0:1]
    d1 = den[:, 1:2]
    col = lax.broadcasted_iota(jnp.int32, (NP, HID), 1)
    divv = jnp.where(col < C, d0, d1)
    out = acc / (divv + 1e-16) + b_ref[...]
    mu = out.mean(axis=1, keepdims=True)
    var = ((out - mu) ** 2).mean(axis=1, keepdims=True)
    ln = (out - mu) / jnp.sqrt(var + 1e-5) * g_ref[...] + be_ref[...]
    return ln


def _tc_mid_body(acc_ref, den_ref, b_ref, g_ref, be_ref, w2_ref,
                 asrc_ref, adst_ref,
                 x1_ref, h_ref, tabs_ref, mv_ref):
    ln = _combine(acc_ref, den_ref, b_ref, g_ref, be_ref)
    x1 = jnp.where(ln > 0.0, ln, jnp.exp(ln) - 1.0)
    x1_ref[...] = x1
    _tc_head_body(HID, x1_ref, w2_ref, asrc_ref, adst_ref,
                  h_ref, tabs_ref, mv_ref)


def _tc_mid(acc1, den1, b1, g1, be1, w2, asrc2, adst2):
    return pl.pallas_call(
        _tc_mid_body,
        out_shape=[
            jax.ShapeDtypeStruct((NP, HID), jnp.float32),
            jax.ShapeDtypeStruct((NP, HID), jnp.float32),
            jax.ShapeDtypeStruct((NP, 4), jnp.float32),
            jax.ShapeDtypeStruct((1, 32), jnp.float32),
        ],
    )(acc1, den1, b1, g1, be1, w2, asrc2, adst2)


def _tc_fin_body(acc_ref, den_ref, b_ref, g_ref, be_ref, x1_ref, wa_ref,
                 ba_ref, out_ref):
    ln = _combine(acc_ref, den_ref, b_ref, g_ref, be_ref)
    x2 = ln + x1_ref[...]
    x2 = jnp.where(x2 > 0.0, x2, jnp.exp(x2) - 1.0)
    logits = jnp.dot(x2, wa_ref[...], preferred_element_type=jnp.float32)
    logits = logits + ba_ref[...]
    row = lax.broadcasted_iota(jnp.int32, (NP, 1), 0)
    logits = jnp.where(row < N, logits, NEG)
    m = jnp.max(logits)
    wgt = jnp.exp(logits - m)
    z = jnp.sum(wgt)
    attw = wgt / z
    out_ref[...] = jnp.sum(x2 * attw, axis=0, keepdims=True)


def _tc_fin(acc2, den2, b2, g2, be2, x1, wa, ba):
    return pl.pallas_call(
        _tc_fin_body,
        out_shape=jax.ShapeDtypeStruct((1, HID), jnp.float32),
    )(acc2, den2, b2, g2, be2, x1, wa, ba)


def kernel(x, edge_index, W1, a_src1, a_dst1, b1, g1, be1, W2, a_src2,
           a_dst2, b2, g2, be2, Wa, ba):
    x_pad = jnp.pad(x, ((0, NP - N), (0, 0)))
    loop = jnp.arange(N, dtype=jnp.int32)
    padv = jnp.full((TOT - 320000 - N,), SENT, dtype=jnp.int32)
    srcp = jnp.concatenate([edge_index[0].astype(jnp.int32), loop, padv])
    dstp = jnp.concatenate([edge_index[1].astype(jnp.int32), loop, padv])
    srcp = srcp.reshape(NTILES, NCH, K)
    dstp = dstp.reshape(NTILES, NCH, K)

    h1, tabs1, mv1 = _tc_head(x_pad, W1, a_src1.reshape(1, HID),
                              a_dst1.reshape(1, HID), 128)
    acc1, den1 = _sc_edge(tabs1.reshape(4 * NP), mv1.reshape(32), srcp, dstp, h1)
    den1 = _denred(den1.reshape(NTILES, 158, 128)).reshape(NP, 2)
    x1, h2, tabs2, mv2 = _tc_mid(acc1, den1,
                                 b1.reshape(1, HID), g1.reshape(1, HID),
                                 be1.reshape(1, HID), W2,
                                 a_src2.reshape(1, HID),
                                 a_dst2.reshape(1, HID))
    acc2, den2 = _sc_edge(tabs2.reshape(4 * NP), mv2.reshape(32), srcp, dstp, h2)
    den2 = _denred(den2.reshape(NTILES, 158, 128)).reshape(NP, 2)
    pooled = _tc_fin(acc2, den2,
                     b2.reshape(1, HID), g2.reshape(1, HID),
                     be2.reshape(1, HID), x1, Wa, ba.reshape(1, 1))
    return pooled.reshape(HID)


# speedup vs baseline: 128.1285x; 1.0965x over previous
import functools

import jax
import jax.numpy as jnp
from jax import lax
from jax.experimental import pallas as pl
from jax.experimental.pallas import tpu as pltpu
from jax.experimental.pallas import tpu_sc as plsc

N = 10000
NP = 10112
SENT = N
HEADS = 2
C = 32
HID = 64
NTILES = 32
K = 64
NCH = 162
EPT = NCH * K
TOT = NTILES * EPT
RPT = NP // 16
NEG = -1e30


def _tc_head_body(din, x_ref, w_ref, asrc_ref, adst_ref,
                  h_ref, tabs_ref, mv_ref):
    x = x_ref[...]
    w = w_ref[...]
    h = jnp.dot(x, w, preferred_element_type=jnp.float32)
    h_ref[...] = h
    av = asrc_ref[...]
    bv = adst_ref[...]
    ps = h * av
    pd = h * bv
    as0 = ps[:, :C].sum(axis=1, keepdims=True)
    as1 = ps[:, C:].sum(axis=1, keepdims=True)
    ad0 = pd[:, :C].sum(axis=1, keepdims=True)
    ad1 = pd[:, C:].sum(axis=1, keepdims=True)
    row = lax.broadcasted_iota(jnp.int32, (NP, 1), 0)
    valid = row < N
    as0 = jnp.where(valid, as0, NEG)
    as1 = jnp.where(valid, as1, NEG)
    ad0 = jnp.where(valid, ad0, NEG)
    ad1 = jnp.where(valid, ad1, NEG)
    tabs_ref[...] = jnp.concatenate([as0, as1, ad0, ad1], axis=1)
    m0 = jnp.max(as0) + jnp.max(ad0)
    m1 = jnp.max(as1) + jnp.max(ad1)
    col = lax.broadcasted_iota(jnp.int32, (1, 32), 1)
    mv_ref[...] = jnp.where(col < 16, m0, m1)


def _tc_head(x_pad, w, asrc, adst, din):
    return pl.pallas_call(
        functools.partial(_tc_head_body, din),
        out_shape=[
            jax.ShapeDtypeStruct((NP, HID), jnp.float32),
            jax.ShapeDtypeStruct((NP, 4), jnp.float32),
            jax.ShapeDtypeStruct((1, 32), jnp.float32),
        ],
    )(x_pad, w, asrc, adst)


def _sc_edge_body(tabs_hbm, mv_hbm, src_hbm, dst_hbm, h_hbm,
                  acc_hbm, den_hbm,
                  tabs_v, sidx_v, didx_v, didxc_v, den_v, rows_v, exb_v, mv_v,
                  acc_sh, sem, sem2):
    c = lax.axis_index("c")
    s = lax.axis_index("s")
    wid = c * 16 + s

    pltpu.sync_copy(tabs_hbm, tabs_v)
    pltpu.sync_copy(mv_hbm, mv_v)
    pltpu.sync_copy(src_hbm.at[wid], sidx_v)
    pltpu.sync_copy(dst_hbm.at[wid], didx_v)

    zv = jnp.zeros((16,), jnp.float32)

    def zden(i, carry):
        den_v[pl.ds(i * 16, 16)] = zv
        return carry

    lax.fori_loop(0, (2 * NP) // 16, zden, 0)

    def zrow(i, carry):
        rows_v[0, i, pl.ds(0, 16)] = zv
        rows_v[0, i, pl.ds(16, 16)] = zv
        rows_v[0, i, pl.ds(32, 16)] = zv
        rows_v[0, i, pl.ds(48, 16)] = zv
        return carry

    lax.fori_loop(0, K, zrow, 0)

    base = s * RPT
    for i in range(9):
        pltpu.sync_copy(rows_v.at[0, pl.ds(0, 64)],
                        acc_sh.at[pl.ds(base + i * 64, 64)])
    pltpu.sync_copy(rows_v.at[0, pl.ds(0, 56)],
                    acc_sh.at[pl.ds(base + 576, 56)])
    plsc.subcore_barrier()

    m0 = mv_v[pl.ds(0, 16)]
    m1 = mv_v[pl.ds(16, 16)]
    sems = (sem, sem2)

    pltpu.async_copy(h_hbm.at[sidx_v.at[0]], rows_v.at[0], sems[0])

    def chunk(j, carry):
        def alpha16(t, inner):
            sv = sidx_v[j, pl.ds(t * 16, 16)]
            dv = didx_v[j, pl.ds(t * 16, 16)]
            didxc_v[pl.ds(t * 16, 16)] = dv
            s4 = sv * 4
            d4 = dv * 4
            as0 = plsc.load_gather(tabs_v, [s4])
            as1 = plsc.load_gather(tabs_v, [s4 + 1])
            ad0 = plsc.load_gather(tabs_v, [d4 + 2])
            ad1 = plsc.load_gather(tabs_v, [d4 + 3])
            a0 = as0 + ad0
            a1 = as1 + ad1
            a0 = jnp.where(a0 > 0.0, a0, a0 * 0.2)
            a1 = jnp.where(a1 > 0.0, a1, a1 * 0.2)
            e0 = jnp.exp(a0 - m0)
            e1 = jnp.exp(a1 - m1)
            d2 = dv * 2
            plsc.addupdate_scatter(den_v, [d2], e0)
            plsc.addupdate_scatter(den_v, [d2 + 1], e1)
            exb_v[pl.ds(t * 16, 16)] = e0
            exb_v[pl.ds(K + t * 16, 16)] = e1
            return inner

        def halfbody(b):
            @pl.when(j + 1 < NCH)
            def _():
                pltpu.async_copy(h_hbm.at[sidx_v.at[j + 1]],
                                 rows_v.at[1 - b], sems[1 - b])

            lax.fori_loop(0, K // 16, alpha16, 0)
            pltpu.make_async_copy(h_hbm.at[sidx_v.at[j]],
                                  rows_v.at[b], sems[b]).wait()

            @plsc.parallel_loop(0, K, unroll=4)
            def mulrow(r):
                s0 = plsc.load_gather(exb_v,
                                      [jnp.full((16,), 0, jnp.int32) + r])
                s1 = plsc.load_gather(exb_v,
                                      [jnp.full((16,), K, jnp.int32) + r])
                rows_v[b, r, pl.ds(0, 16)] = rows_v[b, r, pl.ds(0, 16)] * s0
                rows_v[b, r, pl.ds(16, 16)] = rows_v[b, r, pl.ds(16, 16)] * s0
                rows_v[b, r, pl.ds(32, 16)] = rows_v[b, r, pl.ds(32, 16)] * s1
                rows_v[b, r, pl.ds(48, 16)] = rows_v[b, r, pl.ds(48, 16)] * s1
            pltpu.sync_copy(rows_v.at[b], acc_sh.at[didxc_v], add=True)

        @pl.when(j % 2 == 0)
        def _():
            halfbody(0)

        @pl.when(j % 2 == 1)
        def _():
            halfbody(1)

        return carry

    lax.fori_loop(0, NCH, chunk, 0)

    pltpu.sync_copy(den_v, den_hbm.at[wid])
    plsc.subcore_barrier()
    pltpu.sync_copy(acc_sh.at[pl.ds(base, RPT)],
                    acc_hbm.at[c, pl.ds(base, RPT)])


def _sc_edge(tabs, mv, srcp, dstp, h_pad):
    mesh = plsc.VectorSubcoreMesh(core_axis_name="c", subcore_axis_name="s")
    fn = functools.partial(
        pl.kernel,
        mesh=mesh,
        compiler_params=pltpu.CompilerParams(
            needs_layout_passes=False, use_tc_tiling_on_sc=False),
        out_type=[
            jax.ShapeDtypeStruct((2, NP, HID), jnp.float32),
            jax.ShapeDtypeStruct((NTILES, 2 * NP), jnp.float32),
        ],
        scratch_types=[
            pltpu.VMEM((4 * NP,), jnp.float32),
            pltpu.VMEM((NCH, K), jnp.int32),
            pltpu.VMEM((NCH, K), jnp.int32),
            pltpu.VMEM((K,), jnp.int32),
            pltpu.VMEM((2 * NP,), jnp.float32),
            pltpu.VMEM((2, K, HID), jnp.float32),
            pltpu.VMEM((2 * K,), jnp.float32),
            pltpu.VMEM((32,), jnp.float32),
            pltpu.VMEM_SHARED((NP, HID), jnp.float32),
            pltpu.SemaphoreType.DMA,
            pltpu.SemaphoreType.DMA,
        ],
    )(_sc_edge_body)
    return fn(tabs, mv, srcp, dstp, h_pad)


def _denred_body(den_ref, out_ref):
    t = pl.program_id(0)

    @pl.when(t == 0)
    def _():
        out_ref[...] = jnp.zeros_like(out_ref)

    out_ref[...] += den_ref[0]


def _denred(den):
    return pl.pallas_call(
        _denred_body,
        grid=(NTILES,),
        in_specs=[pl.BlockSpec((1, 158, 128), lambda t: (t, 0, 0))],
        out_specs=pl.BlockSpec((158, 128), lambda t: (0, 0)),
        out_shape=jax.ShapeDtypeStruct((158, 128), jnp.float32),
    )(den)


def _combine(acc_ref, den_ref, b_ref, g_ref, be_ref):
    acc = acc_ref[0] + acc_ref[1]
    den = den_ref[...]
    d0 = den[:, 0:1]
    d1 = den[:, 1:2]
    col = lax.broadcasted_iota(jnp.int32, (NP, HID), 1)
    divv = jnp.where(col < C, d0, d1)
    out = acc / (divv + 1e-16) + b_ref[...]
    mu = out.mean(axis=1, keepdims=True)
    var = ((out - mu) ** 2).mean(axis=1, keepdims=True)
    ln = (out - mu) / jnp.sqrt(var + 1e-5) * g_ref[...] + be_ref[...]
    return ln


def _tc_mid_body(acc_ref, den_ref, b_ref, g_ref, be_ref, w2_ref,
                 asrc_ref, adst_ref,
                 x1_ref, h_ref, tabs_ref, mv_ref):
    ln = _combine(acc_ref, den_ref, b_ref, g_ref, be_ref)
    x1 = jnp.where(ln > 0.0, ln, jnp.exp(ln) - 1.0)
    x1_ref[...] = x1
    _tc_head_body(HID, x1_ref, w2_ref, asrc_ref, adst_ref,
                  h_ref, tabs_ref, mv_ref)


def _tc_mid(acc1, den1, b1, g1, be1, w2, asrc2, adst2):
    return pl.pallas_call(
        _tc_mid_body,
        out_shape=[
            jax.ShapeDtypeStruct((NP, HID), jnp.float32),
            jax.ShapeDtypeStruct((NP, HID), jnp.float32),
            jax.ShapeDtypeStruct((NP, 4), jnp.float32),
            jax.ShapeDtypeStruct((1, 32), jnp.float32),
        ],
    )(acc1, den1, b1, g1, be1, w2, asrc2, adst2)


def _tc_fin_body(acc_ref, den_ref, b_ref, g_ref, be_ref, x1_ref, wa_ref,
                 ba_ref, out_ref):
    ln = _combine(acc_ref, den_ref, b_ref, g_ref, be_ref)
    x2 = ln + x1_ref[...]
    x2 = jnp.where(x2 > 0.0, x2, jnp.exp(x2) - 1.0)
    logits = jnp.dot(x2, wa_ref[...], preferred_element_type=jnp.float32)
    logits = logits + ba_ref[...]
    row = lax.broadcasted_iota(jnp.int32, (NP, 1), 0)
    logits = jnp.where(row < N, logits, NEG)
    m = jnp.max(logits)
    wgt = jnp.exp(logits - m)
    z = jnp.sum(wgt)
    attw = wgt / z
    out_ref[...] = jnp.sum(x2 * attw, axis=0, keepdims=True)


def _tc_fin(acc2, den2, b2, g2, be2, x1, wa, ba):
    return pl.pallas_call(
        _tc_fin_body,
        out_shape=jax.ShapeDtypeStruct((1, HID), jnp.float32),
    )(acc2, den2, b2, g2, be2, x1, wa, ba)


def kernel(x, edge_index, W1, a_src1, a_dst1, b1, g1, be1, W2, a_src2,
           a_dst2, b2, g2, be2, Wa, ba):
    x_pad = jnp.pad(x, ((0, NP - N), (0, 0)))
    loop = jnp.arange(N, dtype=jnp.int32)
    padv = jnp.full((TOT - 320000 - N,), SENT, dtype=jnp.int32)
    srcp = jnp.concatenate([edge_index[0].astype(jnp.int32), loop, padv])
    dstp = jnp.concatenate([edge_index[1].astype(jnp.int32), loop, padv])
    srcp = srcp.reshape(NTILES, NCH, K)
    dstp = dstp.reshape(NTILES, NCH, K)

    h1, tabs1, mv1 = _tc_head(x_pad, W1, a_src1.reshape(1, HID),
                              a_dst1.reshape(1, HID), 128)
    acc1, den1 = _sc_edge(tabs1.reshape(4 * NP), mv1.reshape(32), srcp, dstp, h1)
    den1 = _denred(den1.reshape(NTILES, 158, 128)).reshape(NP, 2)
    x1, h2, tabs2, mv2 = _tc_mid(acc1, den1,
                                 b1.reshape(1, HID), g1.reshape(1, HID),
                                 be1.reshape(1, HID), W2,
                                 a_src2.reshape(1, HID),
                                 a_dst2.reshape(1, HID))
    acc2, den2 = _sc_edge(tabs2.reshape(4 * NP), mv2.reshape(32), srcp, dstp, h2)
    den2 = _denred(den2.reshape(NTILES, 158, 128)).reshape(NP, 2)
    pooled = _tc_fin(acc2, den2,
                     b2.reshape(1, HID), g2.reshape(1, HID),
                     be2.reshape(1, HID), x1, Wa, ba.reshape(1, 1))
    return pooled.reshape(HID)
